# Initial kernel scaffold; baseline (speedup 1.0000x reference)
import jax
import jax.numpy as jnp
from jax.experimental import pallas as pl


def kernel(x, common_idx_flat, dw_w, pw_w, bn_gamma, bn_beta):
    raise NotImplementedError("write your pallas kernel here")


# speedup vs baseline: 3.1975x; 3.1975x over previous
import functools

import jax
import jax.numpy as jnp
from jax import lax
from jax.experimental import pallas as pl
from jax.experimental.pallas import tpu as pltpu
from jax.experimental.pallas import tpu_sc as plsc

B, C, N, K = 1, 128, 10000, 32
OUT = 128
EPS = 1e-5
E = N * K

NW = 32
EPW = E // NW
CHUNK = 80
NCHUNK = EPW // CHUNK

NB = 200
NBLK = N // NB
EB = NB * K


def _sc_gather(xt, idx3):
    mesh = plsc.VectorSubcoreMesh(core_axis_name="c", subcore_axis_name="s")
    info = plsc.get_sparse_core_info()
    nc = info.num_cores

    @functools.partial(
        pl.kernel,
        mesh=mesh,
        out_type=jax.ShapeDtypeStruct((E, C), jnp.float32),
        scratch_types=[
            pltpu.VMEM((NCHUNK, CHUNK), jnp.int32),
            pltpu.VMEM((CHUNK, C), jnp.float32),
            pltpu.SemaphoreType.DMA,
        ],
    )
    def gather_kernel(xt_hbm, idx_hbm, out_hbm, idx_v, rows_v, sem):
        wid = lax.axis_index("s") * nc + lax.axis_index("c")
        pltpu.sync_copy(idx_hbm.at[wid], idx_v)
        base = wid * EPW

        def body(j, carry):
            pltpu.async_copy(xt_hbm.at[idx_v.at[j]], rows_v, sem).wait()
            pltpu.sync_copy(rows_v, out_hbm.at[pl.ds(base + j * CHUNK, CHUNK)])
            return carry

        lax.fori_loop(0, NCHUNK, body, 0)

    return gather_kernel(xt, idx3)


def _stats_body(g_ref, x_ref, o_ref):
    i = pl.program_id(0)

    @pl.when(i == 0)
    def _():
        o_ref[...] = jnp.zeros_like(o_ref)

    g = g_ref[...]
    xb = x_ref[...]
    s_node = jnp.sum(g.reshape(NB, K, C), axis=1)
    sf = jnp.sum(s_node, axis=0, keepdims=True)
    sf2 = jnp.sum(g * g, axis=0, keepdims=True)
    scross = jnp.sum(s_node * xb, axis=0, keepdims=True)
    sx = jnp.sum(xb, axis=0, keepdims=True)
    sx2 = jnp.sum(xb * xb, axis=0, keepdims=True)
    pad = jnp.zeros((3, C), jnp.float32)
    o_ref[...] += jnp.concatenate([sf, sf2, scross, sx, sx2, pad], axis=0)


def _main_body(g_ref, x_ref, s_ref, dw_ref, gam_ref, bet_ref, pw1_ref,
               pw2_ref, o_ref):
    s = s_ref[...]
    sf, sf2 = s[0:1], s[1:2]
    scross, sx, sx2 = s[2:3], s[3:4], s[4:5]
    dw1, dw2 = dw_ref[0:1], dw_ref[1:2]
    g1, g2 = gam_ref[0:1], gam_ref[1:2]
    be1, be2 = bet_ref[0:1], bet_ref[1:2]

    fe = jnp.float32(E)
    fn = jnp.float32(N)
    sd = sf - K * sx
    sd2 = sf2 - 2.0 * scross + K * sx2
    md = sd / fe
    vd = sd2 / fe - md * md
    rst1 = lax.rsqrt(dw1 * dw1 * vd + EPS)
    a1 = g1 * dw1 * rst1
    b1 = be1 - g1 * dw1 * md * rst1
    mx = sx / fn
    vx = sx2 / fn - mx * mx
    rst2 = lax.rsqrt(dw2 * dw2 * vx + EPS)
    a2 = g2 * dw2 * rst2
    b2 = be2 - g2 * dw2 * mx * rst2

    xb = x_ref[...]
    g = g_ref[...]
    d3 = g.reshape(NB, K, C) - xb[:, None, :]
    z = d3 * a1[None] + b1[None]
    act = jnp.maximum(z, 0.2 * z)
    y = lax.dot_general(act.reshape(EB, C), pw1_ref[...],
                        (((1,), (1,)), ((), ())),
                        preferred_element_type=jnp.float32)
    ym = jnp.max(y.reshape(NB, K, OUT), axis=1)
    z2 = xb * a2 + b2
    act2 = jnp.maximum(z2, 0.2 * z2)
    y2 = lax.dot_general(act2, pw2_ref[...],
                         (((1,), (1,)), ((), ())),
                         preferred_element_type=jnp.float32)
    o_ref[...] = ym + y2


def _stats_call(g, xt):
    return pl.pallas_call(
        _stats_body,
        grid=(NBLK,),
        in_specs=[
            pl.BlockSpec((EB, C), lambda i: (i, 0)),
            pl.BlockSpec((NB, C), lambda i: (i, 0)),
        ],
        out_specs=pl.BlockSpec((8, C), lambda i: (0, 0)),
        out_shape=jax.ShapeDtypeStruct((8, C), jnp.float32),
    )(g, xt)


def _main_call(g, xt, sums, dw2c, gam2c, bet2c, pw1, pw2):
    const = lambda i: (0, 0)
    return pl.pallas_call(
        _main_body,
        grid=(NBLK,),
        in_specs=[
            pl.BlockSpec((EB, C), lambda i: (i, 0)),
            pl.BlockSpec((NB, C), lambda i: (i, 0)),
            pl.BlockSpec((8, C), const),
            pl.BlockSpec((2, C), const),
            pl.BlockSpec((2, C), const),
            pl.BlockSpec((2, C), const),
            pl.BlockSpec((OUT, C), const),
            pl.BlockSpec((OUT, C), const),
        ],
        out_specs=pl.BlockSpec((NB, OUT), lambda i: (i, 0)),
        out_shape=jax.ShapeDtypeStruct((N, OUT), jnp.float32),
    )(g, xt, sums, dw2c, gam2c, bet2c, pw1, pw2)


def kernel(x, common_idx_flat, dw_w, pw_w, bn_gamma, bn_beta):
    xt = jnp.transpose(x[0])
    idx3 = common_idx_flat.astype(jnp.int32).reshape(NW, NCHUNK, CHUNK)
    g = _sc_gather(xt, idx3)
    sums = _stats_call(g, xt)
    out_nc = _main_call(
        g, xt, sums,
        dw_w.reshape(2, C), bn_gamma.reshape(2, C), bn_beta.reshape(2, C),
        pw_w[:, :C], pw_w[:, C:],
    )
    return jnp.transpose(out_nc)[None]


# speedup vs baseline: 4.0580x; 1.2691x over previous
import functools

import jax
import jax.numpy as jnp
from jax import lax
from jax.experimental import pallas as pl
from jax.experimental.pallas import tpu as pltpu
from jax.experimental.pallas import tpu_sc as plsc

B, C, N, K = 1, 128, 10000, 32
OUT = 128
EPS = 1e-5
E = N * K

NW = 32
EPW = E // NW
CHUNK = 80
NCHUNK = EPW // CHUNK

NB = 200
NBLK = N // NB
EB = NB * K


def _sc_gather(xt, idx3):
    mesh = plsc.VectorSubcoreMesh(core_axis_name="c", subcore_axis_name="s")
    info = plsc.get_sparse_core_info()
    nc = info.num_cores

    @functools.partial(
        pl.kernel,
        mesh=mesh,
        out_type=jax.ShapeDtypeStruct((E, C), jnp.float32),
        scratch_types=[
            pltpu.VMEM((NCHUNK, CHUNK), jnp.int32),
            pltpu.VMEM((CHUNK, C), jnp.float32),
            pltpu.VMEM((CHUNK, C), jnp.float32),
            pltpu.SemaphoreType.DMA,
            pltpu.SemaphoreType.DMA,
            pltpu.SemaphoreType.DMA,
            pltpu.SemaphoreType.DMA,
        ],
    )
    def gather_kernel(xt_hbm, idx_hbm, out_hbm, idx_v, buf0, buf1,
                      gs0, gs1, ws0, ws1):
        wid = lax.axis_index("s") * nc + lax.axis_index("c")
        pltpu.sync_copy(idx_hbm.at[wid], idx_v)
        base = wid * EPW

        def gather(c, buf, sem):
            return pltpu.make_async_copy(xt_hbm.at[idx_v.at[c]], buf, sem)

        def write(c, buf, sem):
            dst = out_hbm.at[pl.ds(base + c * CHUNK, CHUNK)]
            return pltpu.make_async_copy(buf, dst, sem)

        gather(0, buf0, gs0).start()

        def pair(i, carry):
            c0 = 2 * i
            c1 = c0 + 1

            @pl.when(i > 0)
            def _():
                write(c1 - 2, buf1, ws1).wait()
            gather(c1, buf1, gs1).start()
            gather(c0, buf0, gs0).wait()
            write(c0, buf0, ws0).start()
            write(c0, buf0, ws0).wait()
            gather(c0 + 2, buf0, gs0).start()
            gather(c1, buf1, gs1).wait()
            write(c1, buf1, ws1).start()
            return carry

        lax.fori_loop(0, (NCHUNK - 1) // 2, pair, 0)
        last = NCHUNK - 1
        write(last - 1, buf1, ws1).wait()
        gather(last, buf0, gs0).wait()
        write(last, buf0, ws0).start()
        write(last, buf0, ws0).wait()

    return gather_kernel(xt, idx3)


def _stats_body(g_ref, x_ref, o_ref):
    i = pl.program_id(0)

    @pl.when(i == 0)
    def _():
        o_ref[...] = jnp.zeros_like(o_ref)

    g = g_ref[...]
    xb = x_ref[...]
    s_node = jnp.sum(g.reshape(NB, K, C), axis=1)
    sf = jnp.sum(s_node, axis=0, keepdims=True)
    sf2 = jnp.sum(g * g, axis=0, keepdims=True)
    scross = jnp.sum(s_node * xb, axis=0, keepdims=True)
    sx = jnp.sum(xb, axis=0, keepdims=True)
    sx2 = jnp.sum(xb * xb, axis=0, keepdims=True)
    pad = jnp.zeros((3, C), jnp.float32)
    o_ref[...] += jnp.concatenate([sf, sf2, scross, sx, sx2, pad], axis=0)


def _main_body(g_ref, x_ref, s_ref, dw_ref, gam_ref, bet_ref, pw1_ref,
               pw2_ref, o_ref):
    s = s_ref[...]
    sf, sf2 = s[0:1], s[1:2]
    scross, sx, sx2 = s[2:3], s[3:4], s[4:5]
    dw1, dw2 = dw_ref[0:1], dw_ref[1:2]
    g1, g2 = gam_ref[0:1], gam_ref[1:2]
    be1, be2 = bet_ref[0:1], bet_ref[1:2]

    fe = jnp.float32(E)
    fn = jnp.float32(N)
    sd = sf - K * sx
    sd2 = sf2 - 2.0 * scross + K * sx2
    md = sd / fe
    vd = sd2 / fe - md * md
    rst1 = lax.rsqrt(dw1 * dw1 * vd + EPS)
    a1 = g1 * dw1 * rst1
    b1 = be1 - g1 * dw1 * md * rst1
    mx = sx / fn
    vx = sx2 / fn - mx * mx
    rst2 = lax.rsqrt(dw2 * dw2 * vx + EPS)
    a2 = g2 * dw2 * rst2
    b2 = be2 - g2 * dw2 * mx * rst2

    xb = x_ref[...]
    g = g_ref[...]
    d3 = g.reshape(NB, K, C) - xb[:, None, :]
    z = d3 * a1[None] + b1[None]
    act = jnp.maximum(z, 0.2 * z)
    y = lax.dot_general(act.reshape(EB, C), pw1_ref[...],
                        (((1,), (1,)), ((), ())),
                        preferred_element_type=jnp.float32)
    ym = jnp.max(y.reshape(NB, K, OUT), axis=1)
    z2 = xb * a2 + b2
    act2 = jnp.maximum(z2, 0.2 * z2)
    y2 = lax.dot_general(act2, pw2_ref[...],
                         (((1,), (1,)), ((), ())),
                         preferred_element_type=jnp.float32)
    o_ref[...] = ym + y2


def _stats_call(g, xt):
    return pl.pallas_call(
        _stats_body,
        grid=(NBLK,),
        in_specs=[
            pl.BlockSpec((EB, C), lambda i: (i, 0)),
            pl.BlockSpec((NB, C), lambda i: (i, 0)),
        ],
        out_specs=pl.BlockSpec((8, C), lambda i: (0, 0)),
        out_shape=jax.ShapeDtypeStruct((8, C), jnp.float32),
    )(g, xt)


def _main_call(g, xt, sums, dw2c, gam2c, bet2c, pw1, pw2):
    const = lambda i: (0, 0)
    return pl.pallas_call(
        _main_body,
        grid=(NBLK,),
        in_specs=[
            pl.BlockSpec((EB, C), lambda i: (i, 0)),
            pl.BlockSpec((NB, C), lambda i: (i, 0)),
            pl.BlockSpec((8, C), const),
            pl.BlockSpec((2, C), const),
            pl.BlockSpec((2, C), const),
            pl.BlockSpec((2, C), const),
            pl.BlockSpec((OUT, C), const),
            pl.BlockSpec((OUT, C), const),
        ],
        out_specs=pl.BlockSpec((NB, OUT), lambda i: (i, 0)),
        out_shape=jax.ShapeDtypeStruct((N, OUT), jnp.float32),
    )(g, xt, sums, dw2c, gam2c, bet2c, pw1, pw2)


def kernel(x, common_idx_flat, dw_w, pw_w, bn_gamma, bn_beta):
    xt = jnp.transpose(x[0])
    idx3 = common_idx_flat.astype(jnp.int32).reshape(NW, NCHUNK, CHUNK)
    g = _sc_gather(xt, idx3)
    sums = _stats_call(g, xt)
    out_nc = _main_call(
        g, xt, sums,
        dw_w.reshape(2, C), bn_gamma.reshape(2, C), bn_beta.reshape(2, C),
        pw_w[:, :C], pw_w[:, C:],
    )
    return jnp.transpose(out_nc)[None]


# speedup vs baseline: 5.3150x; 1.3098x over previous
import functools

import jax
import jax.numpy as jnp
import numpy as np
from jax import lax
from jax.experimental import pallas as pl
from jax.experimental.pallas import tpu as pltpu
from jax.experimental.pallas import tpu_sc as plsc

B, C, N, K = 1, 128, 10000, 32
OUT = 128
EPS = 1e-5
E = N * K

NW = 32
GP = 4
CH = GP * K
NG = N // GP
GPW0 = 78
EXTRA = (NG - GPW0 * NW) // 2
MAXG = GPW0 + 2
CPAD = 320
NPAD = N + 16

NB = 200
NBLK = N // NB
EB = NB * K

_VSL = [pl.ds(16 * v, 16) for v in range(8)]


def _sc_gather_stats(xt_pad, idx2):
    mesh = plsc.VectorSubcoreMesh(core_axis_name="c", subcore_axis_name="s")
    info = plsc.get_sparse_core_info()
    nc = info.num_cores

    @functools.partial(
        pl.kernel,
        mesh=mesh,
        out_type=[
            jax.ShapeDtypeStruct((E, C), jnp.float32),
            jax.ShapeDtypeStruct((NW * 8, C), jnp.float32),
        ],
        scratch_types=[
            pltpu.VMEM((MAXG, CH), jnp.int32),
            pltpu.VMEM((CPAD, C), jnp.float32),
            pltpu.VMEM((CH, C), jnp.float32),
            pltpu.VMEM((CH, C), jnp.float32),
            pltpu.VMEM((8, C), jnp.float32),
            pltpu.SemaphoreType.DMA,
            pltpu.SemaphoreType.DMA,
            pltpu.SemaphoreType.DMA,
            pltpu.SemaphoreType.DMA,
        ],
    )
    def body(xt_hbm, idx_hbm, gout_hbm, sout_hbm, idx_v, cen_v, buf0, buf1,
             stats_v, gs0, gs1, ws0, ws1):
        wid = lax.axis_index("s") * nc + lax.axis_index("c")
        g0 = GPW0 * wid + 2 * jnp.minimum(wid, EXTRA)
        n = GPW0 + 2 * (wid < EXTRA).astype(jnp.int32)
        pltpu.sync_copy(idx_hbm.at[wid], idx_v)
        pltpu.sync_copy(xt_hbm.at[pl.ds(g0 * GP, CPAD)], cen_v)
        zero16 = jnp.zeros((16,), jnp.float32)
        for r in range(8):
            for sl in _VSL:
                stats_v[r, sl] = zero16

        def gather(c, buf, sem):
            return pltpu.make_async_copy(xt_hbm.at[idx_v.at[c]], buf, sem)

        def write(c, buf, sem):
            dst = gout_hbm.at[pl.ds((g0 + c) * CH, CH)]
            return pltpu.make_async_copy(buf, dst, sem)

        def stats_chunk(buf, c, vmask):
            def node_body(g, _):
                cen = [cen_v[GP * c + g, sl] for sl in _VSL]

                def row_body(k, carry):
                    s, f2 = carry
                    row = [buf[g * K + k, sl] for sl in _VSL]
                    s = tuple(s[v] + row[v] for v in range(8))
                    f2 = tuple(f2[v] + row[v] * row[v] for v in range(8))
                    return (s, f2)

                z8 = (zero16,) * 8
                s, f2 = lax.fori_loop(0, K, row_body, (z8, z8))
                for v in range(8):
                    plsc.addupdate(stats_v.at[0, _VSL[v]], s[v] * vmask)
                    plsc.addupdate(stats_v.at[1, _VSL[v]], f2[v] * vmask)
                    plsc.addupdate(stats_v.at[2, _VSL[v]],
                                   s[v] * cen[v] * vmask)
                return 0

            lax.fori_loop(0, GP, node_body, 0)

        one16 = zero16 + 1.0
        gather(0, buf0, gs0).start()

        def pair(i, carry):
            c0 = 2 * i
            c1 = c0 + 1

            @pl.when(i > 0)
            def _():
                write(c1 - 2, buf1, ws1).wait()

            @pl.when(c1 < n)
            def _():
                gather(c1, buf1, gs1).start()

            gather(c0, buf0, gs0).wait()
            write(c0, buf0, ws0).start()
            stats_chunk(buf0, c0, one16)
            write(c0, buf0, ws0).wait()

            @pl.when(c0 + 2 < n)
            def _():
                gather(c0 + 2, buf0, gs0).start()

            @pl.when(c1 < n)
            def _():
                gather(c1, buf1, gs1).wait()
                write(c1, buf1, ws1).start()

            vmask = jnp.broadcast_to((c1 < n).astype(jnp.float32), (16,))
            stats_chunk(buf1, c1, vmask)
            return carry

        lax.fori_loop(0, (n + 1) // 2, pair, 0)

        @pl.when(n % 2 == 0)
        def _():
            write(n - 1, buf1, ws1).wait()

        pltpu.sync_copy(stats_v, sout_hbm.at[pl.ds(wid * 8, 8)])

    return body(xt_pad, idx2)


def _xstats_body(x_ref, o_ref):
    i = pl.program_id(0)

    @pl.when(i == 0)
    def _():
        o_ref[...] = jnp.zeros_like(o_ref)

    xb = x_ref[...]
    sx = jnp.sum(xb, axis=0, keepdims=True)
    sx2 = jnp.sum(xb * xb, axis=0, keepdims=True)
    pad = jnp.zeros((6, C), jnp.float32)
    o_ref[...] += jnp.concatenate([sx, sx2, pad], axis=0)


def _main_body(g_ref, x_ref, ss_ref, xs_ref, dw_ref, gam_ref, bet_ref,
               pw1_ref, pw2_ref, o_ref):
    tot = jnp.sum(ss_ref[...].reshape(NW, 8, C), axis=0)
    sf, sf2, scross = tot[0:1], tot[1:2], tot[2:3]
    xs = xs_ref[...]
    sx, sx2 = xs[0:1], xs[1:2]
    dw1, dw2 = dw_ref[0:1], dw_ref[1:2]
    g1, g2 = gam_ref[0:1], gam_ref[1:2]
    be1, be2 = bet_ref[0:1], bet_ref[1:2]

    fe = jnp.float32(E)
    fn = jnp.float32(N)
    sd = sf - K * sx
    sd2 = sf2 - 2.0 * scross + K * sx2
    md = sd / fe
    vd = sd2 / fe - md * md
    rst1 = lax.rsqrt(dw1 * dw1 * vd + EPS)
    a1 = g1 * dw1 * rst1
    b1 = be1 - g1 * dw1 * md * rst1
    mx = sx / fn
    vx = sx2 / fn - mx * mx
    rst2 = lax.rsqrt(dw2 * dw2 * vx + EPS)
    a2 = g2 * dw2 * rst2
    b2 = be2 - g2 * dw2 * mx * rst2

    xb = x_ref[...]
    g = g_ref[...]
    d3 = g.reshape(NB, K, C) - xb[:, None, :]
    z = d3 * a1[None] + b1[None]
    act = jnp.maximum(z, 0.2 * z)
    y = lax.dot_general(act.reshape(EB, C), pw1_ref[...],
                        (((1,), (1,)), ((), ())),
                        preferred_element_type=jnp.float32)
    ym = jnp.max(y.reshape(NB, K, OUT), axis=1)
    z2 = xb * a2 + b2
    act2 = jnp.maximum(z2, 0.2 * z2)
    y2 = lax.dot_general(act2, pw2_ref[...],
                         (((1,), (1,)), ((), ())),
                         preferred_element_type=jnp.float32)
    o_ref[...] = ym + y2


def _xstats_call(xt):
    return pl.pallas_call(
        _xstats_body,
        grid=(NBLK,),
        in_specs=[pl.BlockSpec((NB, C), lambda i: (i, 0))],
        out_specs=pl.BlockSpec((8, C), lambda i: (0, 0)),
        out_shape=jax.ShapeDtypeStruct((8, C), jnp.float32),
    )(xt)


def _main_call(g, xt, scstats, xstats, dw2c, gam2c, bet2c, pw1, pw2):
    const = lambda i: (0, 0)
    return pl.pallas_call(
        _main_body,
        grid=(NBLK,),
        in_specs=[
            pl.BlockSpec((EB, C), lambda i: (i, 0)),
            pl.BlockSpec((NB, C), lambda i: (i, 0)),
            pl.BlockSpec((NW * 8, C), const),
            pl.BlockSpec((8, C), const),
            pl.BlockSpec((2, C), const),
            pl.BlockSpec((2, C), const),
            pl.BlockSpec((2, C), const),
            pl.BlockSpec((OUT, C), const),
            pl.BlockSpec((OUT, C), const),
        ],
        out_specs=pl.BlockSpec((NB, OUT), lambda i: (i, 0)),
        out_shape=jax.ShapeDtypeStruct((N, OUT), jnp.float32),
    )(g, xt, scstats, xstats, dw2c, gam2c, bet2c, pw1, pw2)


_G0 = [GPW0 * w + 2 * min(w, EXTRA) for w in range(NW)]
_ROW_IDS = np.asarray(
    [[min(_G0[w] + j, NG - 1) for j in range(MAXG)] for w in range(NW)],
    dtype=np.int32)


def kernel(x, common_idx_flat, dw_w, pw_w, bn_gamma, bn_beta):
    xt = jnp.transpose(x[0])
    xt_pad = jnp.pad(xt, ((0, NPAD - N), (0, 0)))
    idx2 = common_idx_flat.astype(jnp.int32).reshape(NG, CH)
    idx3 = jnp.take(idx2, _ROW_IDS, axis=0)
    g, scstats = _sc_gather_stats(xt_pad, idx3)
    xstats = _xstats_call(xt)
    out_nc = _main_call(
        g, xt, scstats, xstats,
        dw_w.reshape(2, C), bn_gamma.reshape(2, C), bn_beta.reshape(2, C),
        pw_w[:, :C], pw_w[:, C:],
    )
    return jnp.transpose(out_nc)[None]


# speedup vs baseline: 5.4058x; 1.0171x over previous
import functools

import jax
import jax.numpy as jnp
import numpy as np
from jax import lax
from jax.experimental import pallas as pl
from jax.experimental.pallas import tpu as pltpu
from jax.experimental.pallas import tpu_sc as plsc

B, C, N, K = 1, 128, 10000, 32
OUT = 128
EPS = 1e-5
E = N * K

NW = 32
GP = 4
CH = GP * K
NG = N // GP
GPW0 = 78
EXTRA = (NG - GPW0 * NW) // 2
MAXG = GPW0 + 2
CPAD = 320

NB = 200
NBLK = N // NB
EB = NB * K

_VSL = [pl.ds(16 * v, 16) for v in range(8)]


def _sc_gather_stats(xt_pad, idx2):
    mesh = plsc.VectorSubcoreMesh(core_axis_name="c", subcore_axis_name="s")
    info = plsc.get_sparse_core_info()
    nc = info.num_cores

    @functools.partial(
        pl.kernel,
        mesh=mesh,
        out_type=[
            jax.ShapeDtypeStruct((E, C), jnp.float32),
            jax.ShapeDtypeStruct((NW * 8, C), jnp.float32),
        ],
        scratch_types=[
            pltpu.VMEM((MAXG, CH), jnp.int32),
            pltpu.VMEM((CPAD, C), jnp.float32),
            pltpu.VMEM((CH, C), jnp.float32),
            pltpu.VMEM((CH, C), jnp.float32),
            pltpu.VMEM((8, C), jnp.float32),
            pltpu.SemaphoreType.DMA,
            pltpu.SemaphoreType.DMA,
            pltpu.SemaphoreType.DMA,
            pltpu.SemaphoreType.DMA,
        ],
    )
    def body(xt_hbm, idx_hbm, gout_hbm, sout_hbm, idx_v, cen_v, buf0, buf1,
             stats_v, gs0, gs1, ws0, ws1):
        wid = lax.axis_index("s") * nc + lax.axis_index("c")
        extra = jnp.maximum(wid - (NW - EXTRA), 0)
        g0 = GPW0 * wid + 2 * extra
        n = GPW0 + 2 * (wid >= NW - EXTRA).astype(jnp.int32)
        pltpu.sync_copy(idx_hbm.at[wid], idx_v)
        pltpu.sync_copy(xt_hbm.at[pl.ds(g0 * GP, CPAD)], cen_v)
        zero16 = jnp.zeros((16,), jnp.float32)
        for r in range(8):
            for sl in _VSL:
                stats_v[r, sl] = zero16

        def gather(c, buf, sem):
            return pltpu.make_async_copy(xt_hbm.at[idx_v.at[c]], buf, sem)

        def write(c, buf, sem):
            dst = gout_hbm.at[pl.ds((g0 + c) * CH, CH)]
            return pltpu.make_async_copy(buf, dst, sem)

        def stats_chunk(buf, c, vmask):
            def node_body(g, _):
                cen = [cen_v[GP * c + g, sl] for sl in _VSL]

                def row_body(k, carry):
                    s, f2 = carry
                    row = [buf[g * K + k, sl] for sl in _VSL]
                    s = tuple(s[v] + row[v] for v in range(8))
                    f2 = tuple(f2[v] + row[v] * row[v] for v in range(8))
                    return (s, f2)

                z8 = (zero16,) * 8
                s, f2 = lax.fori_loop(0, K, row_body, (z8, z8))
                for v in range(8):
                    plsc.addupdate(stats_v.at[0, _VSL[v]], s[v] * vmask)
                    plsc.addupdate(stats_v.at[1, _VSL[v]], f2[v] * vmask)
                    plsc.addupdate(stats_v.at[2, _VSL[v]],
                                   s[v] * cen[v] * vmask)
                return 0

            lax.fori_loop(0, GP, node_body, 0)

        one16 = zero16 + 1.0
        gather(0, buf0, gs0).start()

        def pair(i, carry):
            c0 = 2 * i
            c1 = c0 + 1

            @pl.when(i > 0)
            def _():
                write(c1 - 2, buf1, ws1).wait()

            @pl.when(c1 < n)
            def _():
                gather(c1, buf1, gs1).start()

            gather(c0, buf0, gs0).wait()
            write(c0, buf0, ws0).start()
            stats_chunk(buf0, c0, one16)
            write(c0, buf0, ws0).wait()

            @pl.when(c0 + 2 < n)
            def _():
                gather(c0 + 2, buf0, gs0).start()

            @pl.when(c1 < n)
            def _():
                gather(c1, buf1, gs1).wait()
                write(c1, buf1, ws1).start()

            vmask = jnp.broadcast_to((c1 < n).astype(jnp.float32), (16,))
            stats_chunk(buf1, c1, vmask)
            return carry

        lax.fori_loop(0, (n + 1) // 2, pair, 0)

        @pl.when(n % 2 == 0)
        def _():
            write(n - 1, buf1, ws1).wait()

        pltpu.sync_copy(stats_v, sout_hbm.at[pl.ds(wid * 8, 8)])

    return body(xt_pad, idx2)


def _xstats_body(x_ref, o_ref):
    i = pl.program_id(0)

    @pl.when(i == 0)
    def _():
        o_ref[...] = jnp.zeros_like(o_ref)

    xb = x_ref[...]
    sx = jnp.sum(xb, axis=0, keepdims=True)
    sx2 = jnp.sum(xb * xb, axis=0, keepdims=True)
    pad = jnp.zeros((6, C), jnp.float32)
    o_ref[...] += jnp.concatenate([sx, sx2, pad], axis=0)


def _main_body(g_ref, x_ref, ss_ref, xs_ref, dw_ref, gam_ref, bet_ref,
               pw1_ref, pw2_ref, o_ref):
    tot = jnp.sum(ss_ref[...].reshape(NW, 8, C), axis=0)
    sf, sf2, scross = tot[0:1], tot[1:2], tot[2:3]
    xs = xs_ref[...]
    sx, sx2 = xs[0:1], xs[1:2]
    dw1, dw2 = dw_ref[0:1], dw_ref[1:2]
    g1, g2 = gam_ref[0:1], gam_ref[1:2]
    be1, be2 = bet_ref[0:1], bet_ref[1:2]

    fe = jnp.float32(E)
    fn = jnp.float32(N)
    sd = sf - K * sx
    sd2 = sf2 - 2.0 * scross + K * sx2
    md = sd / fe
    vd = sd2 / fe - md * md
    rst1 = lax.rsqrt(dw1 * dw1 * vd + EPS)
    a1 = g1 * dw1 * rst1
    b1 = be1 - g1 * dw1 * md * rst1
    mx = sx / fn
    vx = sx2 / fn - mx * mx
    rst2 = lax.rsqrt(dw2 * dw2 * vx + EPS)
    a2 = g2 * dw2 * rst2
    b2 = be2 - g2 * dw2 * mx * rst2

    xb = x_ref[...]
    g = g_ref[...]
    t = b1 - xb * a1
    z = g.reshape(NB, K, C) * a1[None] + t[:, None, :]
    act = jnp.maximum(z, 0.2 * z)
    y = lax.dot_general(act.reshape(EB, C), pw1_ref[...],
                        (((1,), (1,)), ((), ())),
                        preferred_element_type=jnp.float32)
    ym = jnp.max(y.reshape(NB, K, OUT), axis=1)
    z2 = xb * a2 + b2
    act2 = jnp.maximum(z2, 0.2 * z2)
    y2 = lax.dot_general(act2, pw2_ref[...],
                         (((1,), (1,)), ((), ())),
                         preferred_element_type=jnp.float32)
    o_ref[...] = ym + y2


def _xstats_call(xt):
    return pl.pallas_call(
        _xstats_body,
        grid=(NBLK,),
        in_specs=[pl.BlockSpec((NB, C), lambda i: (i, 0))],
        out_specs=pl.BlockSpec((8, C), lambda i: (0, 0)),
        out_shape=jax.ShapeDtypeStruct((8, C), jnp.float32),
    )(xt)


def _main_call(g, xt, scstats, xstats, dw2c, gam2c, bet2c, pw1, pw2):
    const = lambda i: (0, 0)
    return pl.pallas_call(
        _main_body,
        grid=(NBLK,),
        in_specs=[
            pl.BlockSpec((EB, C), lambda i: (i, 0)),
            pl.BlockSpec((NB, C), lambda i: (i, 0)),
            pl.BlockSpec((NW * 8, C), const),
            pl.BlockSpec((8, C), const),
            pl.BlockSpec((2, C), const),
            pl.BlockSpec((2, C), const),
            pl.BlockSpec((2, C), const),
            pl.BlockSpec((OUT, C), const),
            pl.BlockSpec((OUT, C), const),
        ],
        out_specs=pl.BlockSpec((NB, OUT), lambda i: (i, 0)),
        out_shape=jax.ShapeDtypeStruct((N, OUT), jnp.float32),
    )(g, xt, scstats, xstats, dw2c, gam2c, bet2c, pw1, pw2)


_G0 = [GPW0 * w + 2 * max(0, w - (NW - EXTRA)) for w in range(NW)]
_ROW_IDS = np.asarray(
    [[min(_G0[w] + j, NG - 1) for j in range(MAXG)] for w in range(NW)],
    dtype=np.int32)


def kernel(x, common_idx_flat, dw_w, pw_w, bn_gamma, bn_beta):
    xt = jnp.transpose(x[0])
    idx2 = common_idx_flat.astype(jnp.int32).reshape(NG, CH)
    idx3 = jnp.take(idx2, _ROW_IDS, axis=0)
    g, scstats = _sc_gather_stats(xt, idx3)
    xstats = _xstats_call(xt)
    out_nc = _main_call(
        g, xt, scstats, xstats,
        dw_w.reshape(2, C), bn_gamma.reshape(2, C), bn_beta.reshape(2, C),
        pw_w[:, :C], pw_w[:, C:],
    )
    return jnp.transpose(out_nc)[None]


# speedup vs baseline: 5.7711x; 1.0676x over previous
import functools

import jax
import jax.numpy as jnp
import numpy as np
from jax import lax
from jax.experimental import pallas as pl
from jax.experimental.pallas import tpu as pltpu
from jax.experimental.pallas import tpu_sc as plsc

B, C, N, K = 1, 128, 10000, 32
OUT = 128
EPS = 1e-5
E = N * K

NW = 32
GP = 4
CH = GP * K
NG = N // GP
GPW0 = 78
EXTRA = (NG - GPW0 * NW) // 2
MAXG = GPW0 + 2
CPAD = 320

NB = 400
NBLK = N // NB
EB = NB * K

_VSL = [pl.ds(16 * v, 16) for v in range(8)]


def _sc_gather_stats(xt_pad, idx2):
    mesh = plsc.VectorSubcoreMesh(core_axis_name="c", subcore_axis_name="s")
    info = plsc.get_sparse_core_info()
    nc = info.num_cores

    @functools.partial(
        pl.kernel,
        mesh=mesh,
        out_type=[
            jax.ShapeDtypeStruct((E, C), jnp.float32),
            jax.ShapeDtypeStruct((NW * 8, C), jnp.float32),
        ],
        scratch_types=[
            pltpu.VMEM((MAXG, CH), jnp.int32),
            pltpu.VMEM((CPAD, C), jnp.float32),
            pltpu.VMEM((CH, C), jnp.float32),
            pltpu.VMEM((CH, C), jnp.float32),
            pltpu.VMEM((8, C), jnp.float32),
            pltpu.SemaphoreType.DMA,
            pltpu.SemaphoreType.DMA,
            pltpu.SemaphoreType.DMA,
            pltpu.SemaphoreType.DMA,
        ],
    )
    def body(xt_hbm, idx_hbm, gout_hbm, sout_hbm, idx_v, cen_v, buf0, buf1,
             stats_v, gs0, gs1, ws0, ws1):
        wid = lax.axis_index("s") * nc + lax.axis_index("c")
        extra = jnp.maximum(wid - (NW - EXTRA), 0)
        g0 = GPW0 * wid + 2 * extra
        n = GPW0 + 2 * (wid >= NW - EXTRA).astype(jnp.int32)
        pltpu.sync_copy(idx_hbm.at[wid], idx_v)
        pltpu.sync_copy(xt_hbm.at[pl.ds(g0 * GP, CPAD)], cen_v)
        zero16 = jnp.zeros((16,), jnp.float32)
        for r in range(8):
            for sl in _VSL:
                stats_v[r, sl] = zero16

        def gather(c, buf, sem):
            return pltpu.make_async_copy(xt_hbm.at[idx_v.at[c]], buf, sem)

        def write(c, buf, sem):
            dst = gout_hbm.at[pl.ds((g0 + c) * CH, CH)]
            return pltpu.make_async_copy(buf, dst, sem)

        def stats_chunk(buf, c, vmask):
            def node_body(g, _):
                cen = [cen_v[GP * c + g, sl] for sl in _VSL]

                def row_body(k, carry):
                    s, f2 = carry
                    row = [buf[g * K + k, sl] for sl in _VSL]
                    s = tuple(s[v] + row[v] for v in range(8))
                    f2 = tuple(f2[v] + row[v] * row[v] for v in range(8))
                    return (s, f2)

                z8 = (zero16,) * 8
                s, f2 = lax.fori_loop(0, K, row_body, (z8, z8))
                for v in range(8):
                    plsc.addupdate(stats_v.at[0, _VSL[v]], s[v] * vmask)
                    plsc.addupdate(stats_v.at[1, _VSL[v]], f2[v] * vmask)
                    plsc.addupdate(stats_v.at[2, _VSL[v]],
                                   s[v] * cen[v] * vmask)
                return 0

            lax.fori_loop(0, GP, node_body, 0)

        one16 = zero16 + 1.0
        gather(0, buf0, gs0).start()

        def pair(i, carry):
            c0 = 2 * i
            c1 = c0 + 1

            @pl.when(i > 0)
            def _():
                write(c1 - 2, buf1, ws1).wait()

            @pl.when(c1 < n)
            def _():
                gather(c1, buf1, gs1).start()

            gather(c0, buf0, gs0).wait()
            write(c0, buf0, ws0).start()
            stats_chunk(buf0, c0, one16)
            write(c0, buf0, ws0).wait()

            @pl.when(c0 + 2 < n)
            def _():
                gather(c0 + 2, buf0, gs0).start()

            @pl.when(c1 < n)
            def _():
                gather(c1, buf1, gs1).wait()
                write(c1, buf1, ws1).start()

            vmask = jnp.broadcast_to((c1 < n).astype(jnp.float32), (16,))
            stats_chunk(buf1, c1, vmask)
            return carry

        lax.fori_loop(0, (n + 1) // 2, pair, 0)

        @pl.when(n % 2 == 0)
        def _():
            write(n - 1, buf1, ws1).wait()

        pltpu.sync_copy(stats_v, sout_hbm.at[pl.ds(wid * 8, 8)])

    return body(xt_pad, idx2)


def _xstats_body(x_ref, o_ref):
    i = pl.program_id(0)

    @pl.when(i == 0)
    def _():
        o_ref[...] = jnp.zeros_like(o_ref)

    xb = x_ref[...]
    sx = jnp.sum(xb, axis=0, keepdims=True)
    sx2 = jnp.sum(xb * xb, axis=0, keepdims=True)
    pad = jnp.zeros((6, C), jnp.float32)
    o_ref[...] += jnp.concatenate([sx, sx2, pad], axis=0)


def _main_body(g_ref, x_ref, ss_ref, xs_ref, dw_ref, gam_ref, bet_ref,
               pw1_ref, pw2_ref, o_ref):
    tot = jnp.sum(ss_ref[...].reshape(NW, 8, C), axis=0)
    sf, sf2, scross = tot[0:1], tot[1:2], tot[2:3]
    xs = xs_ref[...]
    sx, sx2 = xs[0:1], xs[1:2]
    dw1, dw2 = dw_ref[0:1], dw_ref[1:2]
    g1, g2 = gam_ref[0:1], gam_ref[1:2]
    be1, be2 = bet_ref[0:1], bet_ref[1:2]

    fe = jnp.float32(E)
    fn = jnp.float32(N)
    sd = sf - K * sx
    sd2 = sf2 - 2.0 * scross + K * sx2
    md = sd / fe
    vd = sd2 / fe - md * md
    rst1 = lax.rsqrt(dw1 * dw1 * vd + EPS)
    a1 = g1 * dw1 * rst1
    b1 = be1 - g1 * dw1 * md * rst1
    mx = sx / fn
    vx = sx2 / fn - mx * mx
    rst2 = lax.rsqrt(dw2 * dw2 * vx + EPS)
    a2 = g2 * dw2 * rst2
    b2 = be2 - g2 * dw2 * mx * rst2

    xb = x_ref[...]
    g = g_ref[...]
    t = b1 - xb * a1
    z = g.reshape(NB, K, C) * a1[None] + t[:, None, :]
    act = jnp.maximum(z, 0.2 * z)
    y = lax.dot_general(act.reshape(EB, C), pw1_ref[...],
                        (((1,), (1,)), ((), ())),
                        preferred_element_type=jnp.float32)
    ym = jnp.max(y.reshape(NB, K, OUT), axis=1)
    z2 = xb * a2 + b2
    act2 = jnp.maximum(z2, 0.2 * z2)
    y2 = lax.dot_general(act2, pw2_ref[...],
                         (((1,), (1,)), ((), ())),
                         preferred_element_type=jnp.float32)
    o_ref[...] = ym + y2


def _xstats_call(xt):
    return pl.pallas_call(
        _xstats_body,
        grid=(NBLK,),
        in_specs=[pl.BlockSpec((NB, C), lambda i: (i, 0))],
        out_specs=pl.BlockSpec((8, C), lambda i: (0, 0)),
        out_shape=jax.ShapeDtypeStruct((8, C), jnp.float32),
    )(xt)


def _main_call(g, xt, scstats, xstats, dw2c, gam2c, bet2c, pw1, pw2):
    const = lambda i: (0, 0)
    return pl.pallas_call(
        _main_body,
        grid=(NBLK,),
        in_specs=[
            pl.BlockSpec((EB, C), lambda i: (i, 0)),
            pl.BlockSpec((NB, C), lambda i: (i, 0)),
            pl.BlockSpec((NW * 8, C), const),
            pl.BlockSpec((8, C), const),
            pl.BlockSpec((2, C), const),
            pl.BlockSpec((2, C), const),
            pl.BlockSpec((2, C), const),
            pl.BlockSpec((OUT, C), const),
            pl.BlockSpec((OUT, C), const),
        ],
        out_specs=pl.BlockSpec((NB, OUT), lambda i: (i, 0)),
        out_shape=jax.ShapeDtypeStruct((N, OUT), jnp.float32),
    )(g, xt, scstats, xstats, dw2c, gam2c, bet2c, pw1, pw2)


_G0 = [GPW0 * w + 2 * max(0, w - (NW - EXTRA)) for w in range(NW)]
_ROW_IDS = np.asarray(
    [[min(_G0[w] + j, NG - 1) for j in range(MAXG)] for w in range(NW)],
    dtype=np.int32)


def kernel(x, common_idx_flat, dw_w, pw_w, bn_gamma, bn_beta):
    xt = jnp.transpose(x[0])
    idx2 = common_idx_flat.astype(jnp.int32).reshape(NG, CH)
    idx3 = jnp.take(idx2, _ROW_IDS, axis=0)
    g, scstats = _sc_gather_stats(xt, idx3)
    xstats = _xstats_call(xt)
    out_nc = _main_call(
        g, xt, scstats, xstats,
        dw_w.reshape(2, C), bn_gamma.reshape(2, C), bn_beta.reshape(2, C),
        pw_w[:, :C], pw_w[:, C:],
    )
    return jnp.transpose(out_nc)[None]


# speedup vs baseline: 5.9290x; 1.0274x over previous
import functools

import jax
import jax.numpy as jnp
import numpy as np
from jax import lax
from jax.experimental import pallas as pl
from jax.experimental.pallas import tpu as pltpu
from jax.experimental.pallas import tpu_sc as plsc

B, C, N, K = 1, 128, 10000, 32
OUT = 128
EPS = 1e-5
E = N * K

NW = 32
GP = 4
CH = GP * K
NG = N // GP
GPW0 = 78
EXTRA = (NG - GPW0 * NW) // 2
MAXG = GPW0 + 2
CPAD = 320

NB = 1000
NBLK = N // NB
EB = NB * K

_VSL = [pl.ds(16 * v, 16) for v in range(8)]


def _sc_gather_stats(xt_pad, idx2):
    mesh = plsc.VectorSubcoreMesh(core_axis_name="c", subcore_axis_name="s")
    info = plsc.get_sparse_core_info()
    nc = info.num_cores

    @functools.partial(
        pl.kernel,
        mesh=mesh,
        out_type=[
            jax.ShapeDtypeStruct((E, C), jnp.float32),
            jax.ShapeDtypeStruct((NW * 8, C), jnp.float32),
        ],
        scratch_types=[
            pltpu.VMEM((MAXG, CH), jnp.int32),
            pltpu.VMEM((CPAD, C), jnp.float32),
            pltpu.VMEM((CH, C), jnp.float32),
            pltpu.VMEM((CH, C), jnp.float32),
            pltpu.VMEM((8, C), jnp.float32),
            pltpu.SemaphoreType.DMA,
            pltpu.SemaphoreType.DMA,
            pltpu.SemaphoreType.DMA,
            pltpu.SemaphoreType.DMA,
        ],
    )
    def body(xt_hbm, idx_hbm, gout_hbm, sout_hbm, idx_v, cen_v, buf0, buf1,
             stats_v, gs0, gs1, ws0, ws1):
        wid = lax.axis_index("s") * nc + lax.axis_index("c")
        extra = jnp.maximum(wid - (NW - EXTRA), 0)
        g0 = GPW0 * wid + 2 * extra
        n = GPW0 + 2 * (wid >= NW - EXTRA).astype(jnp.int32)
        pltpu.sync_copy(idx_hbm.at[wid], idx_v)
        pltpu.sync_copy(xt_hbm.at[pl.ds(g0 * GP, CPAD)], cen_v)
        zero16 = jnp.zeros((16,), jnp.float32)
        for r in range(8):
            for sl in _VSL:
                stats_v[r, sl] = zero16

        def gather(c, buf, sem):
            return pltpu.make_async_copy(xt_hbm.at[idx_v.at[c]], buf, sem)

        def write(c, buf, sem):
            dst = gout_hbm.at[pl.ds((g0 + c) * CH, CH)]
            return pltpu.make_async_copy(buf, dst, sem)

        def stats_chunk(buf, c, vmask):
            def node_body(g, _):
                cen = [cen_v[GP * c + g, sl] for sl in _VSL]

                def row_body(k, carry):
                    s, f2 = carry
                    row = [buf[g * K + k, sl] for sl in _VSL]
                    s = tuple(s[v] + row[v] for v in range(8))
                    f2 = tuple(f2[v] + row[v] * row[v] for v in range(8))
                    return (s, f2)

                z8 = (zero16,) * 8
                s, f2 = lax.fori_loop(0, K, row_body, (z8, z8))
                for v in range(8):
                    plsc.addupdate(stats_v.at[0, _VSL[v]], s[v] * vmask)
                    plsc.addupdate(stats_v.at[1, _VSL[v]], f2[v] * vmask)
                    plsc.addupdate(stats_v.at[2, _VSL[v]],
                                   s[v] * cen[v] * vmask)
                return 0

            lax.fori_loop(0, GP, node_body, 0)

        one16 = zero16 + 1.0
        gather(0, buf0, gs0).start()

        def pair(i, carry):
            c0 = 2 * i
            c1 = c0 + 1

            @pl.when(i > 0)
            def _():
                write(c1 - 2, buf1, ws1).wait()

            @pl.when(c1 < n)
            def _():
                gather(c1, buf1, gs1).start()

            gather(c0, buf0, gs0).wait()
            write(c0, buf0, ws0).start()
            stats_chunk(buf0, c0, one16)
            write(c0, buf0, ws0).wait()

            @pl.when(c0 + 2 < n)
            def _():
                gather(c0 + 2, buf0, gs0).start()

            @pl.when(c1 < n)
            def _():
                gather(c1, buf1, gs1).wait()
                write(c1, buf1, ws1).start()

            vmask = jnp.broadcast_to((c1 < n).astype(jnp.float32), (16,))
            stats_chunk(buf1, c1, vmask)
            return carry

        lax.fori_loop(0, (n + 1) // 2, pair, 0)

        @pl.when(n % 2 == 0)
        def _():
            write(n - 1, buf1, ws1).wait()

        pltpu.sync_copy(stats_v, sout_hbm.at[pl.ds(wid * 8, 8)])

    return body(xt_pad, idx2)


def _xstats_body(x_ref, o_ref):
    i = pl.program_id(0)

    @pl.when(i == 0)
    def _():
        o_ref[...] = jnp.zeros_like(o_ref)

    xb = x_ref[...]
    sx = jnp.sum(xb, axis=0, keepdims=True)
    sx2 = jnp.sum(xb * xb, axis=0, keepdims=True)
    pad = jnp.zeros((6, C), jnp.float32)
    o_ref[...] += jnp.concatenate([sx, sx2, pad], axis=0)


def _main_body(g_ref, x_ref, ss_ref, xs_ref, dw_ref, gam_ref, bet_ref,
               pw1_ref, pw2_ref, o_ref):
    tot = jnp.sum(ss_ref[...].reshape(NW, 8, C), axis=0)
    sf, sf2, scross = tot[0:1], tot[1:2], tot[2:3]
    xs = xs_ref[...]
    sx, sx2 = xs[0:1], xs[1:2]
    dw1, dw2 = dw_ref[0:1], dw_ref[1:2]
    g1, g2 = gam_ref[0:1], gam_ref[1:2]
    be1, be2 = bet_ref[0:1], bet_ref[1:2]

    fe = jnp.float32(E)
    fn = jnp.float32(N)
    sd = sf - K * sx
    sd2 = sf2 - 2.0 * scross + K * sx2
    md = sd / fe
    vd = sd2 / fe - md * md
    rst1 = lax.rsqrt(dw1 * dw1 * vd + EPS)
    a1 = g1 * dw1 * rst1
    b1 = be1 - g1 * dw1 * md * rst1
    mx = sx / fn
    vx = sx2 / fn - mx * mx
    rst2 = lax.rsqrt(dw2 * dw2 * vx + EPS)
    a2 = g2 * dw2 * rst2
    b2 = be2 - g2 * dw2 * mx * rst2

    xb = x_ref[...]
    g = g_ref[...]
    t = b1 - xb * a1
    z = g.reshape(NB, K, C) * a1[None] + t[:, None, :]
    act = jnp.maximum(z, 0.2 * z)
    y = lax.dot_general(act.reshape(EB, C), pw1_ref[...],
                        (((1,), (1,)), ((), ())),
                        preferred_element_type=jnp.float32)
    ym = jnp.max(y.reshape(NB, K, OUT), axis=1)
    z2 = xb * a2 + b2
    act2 = jnp.maximum(z2, 0.2 * z2)
    y2 = lax.dot_general(act2, pw2_ref[...],
                         (((1,), (1,)), ((), ())),
                         preferred_element_type=jnp.float32)
    o_ref[...] = ym + y2


def _xstats_call(xt):
    return pl.pallas_call(
        _xstats_body,
        grid=(NBLK,),
        in_specs=[pl.BlockSpec((NB, C), lambda i: (i, 0))],
        out_specs=pl.BlockSpec((8, C), lambda i: (0, 0)),
        out_shape=jax.ShapeDtypeStruct((8, C), jnp.float32),
    )(xt)


def _main_call(g, xt, scstats, xstats, dw2c, gam2c, bet2c, pw1, pw2):
    const = lambda i: (0, 0)
    return pl.pallas_call(
        _main_body,
        grid=(NBLK,),
        in_specs=[
            pl.BlockSpec((EB, C), lambda i: (i, 0)),
            pl.BlockSpec((NB, C), lambda i: (i, 0)),
            pl.BlockSpec((NW * 8, C), const),
            pl.BlockSpec((8, C), const),
            pl.BlockSpec((2, C), const),
            pl.BlockSpec((2, C), const),
            pl.BlockSpec((2, C), const),
            pl.BlockSpec((OUT, C), const),
            pl.BlockSpec((OUT, C), const),
        ],
        out_specs=pl.BlockSpec((NB, OUT), lambda i: (i, 0)),
        out_shape=jax.ShapeDtypeStruct((N, OUT), jnp.float32),
    )(g, xt, scstats, xstats, dw2c, gam2c, bet2c, pw1, pw2)


_G0 = [GPW0 * w + 2 * max(0, w - (NW - EXTRA)) for w in range(NW)]
_ROW_IDS = np.asarray(
    [[min(_G0[w] + j, NG - 1) for j in range(MAXG)] for w in range(NW)],
    dtype=np.int32)


def kernel(x, common_idx_flat, dw_w, pw_w, bn_gamma, bn_beta):
    xt = jnp.transpose(x[0])
    idx2 = common_idx_flat.astype(jnp.int32).reshape(NG, CH)
    idx3 = jnp.take(idx2, _ROW_IDS, axis=0)
    g, scstats = _sc_gather_stats(xt, idx3)
    xstats = _xstats_call(xt)
    out_nc = _main_call(
        g, xt, scstats, xstats,
        dw_w.reshape(2, C), bn_gamma.reshape(2, C), bn_beta.reshape(2, C),
        pw_w[:, :C], pw_w[:, C:],
    )
    return jnp.transpose(out_nc)[None]
